# baseline (speedup 1.0000x reference)
import functools

import jax
import jax.numpy as jnp
from jax import lax
from jax.experimental import pallas as pl
from jax.experimental.pallas import tpu as pltpu
from jax.experimental.pallas import tpu_sc as plsc

VOCAB = 1000
EMB = 64
B = 1024
NCHUNK = 8
CSTRIDE = 1024
PLROWS = NCHUNK * CSTRIDE

NC = 2
NS = 16
NW = NC * NS
RPW = B // NW
G = 16
GROUPS = RPW // G
UNROLL = 8


def _tc_p_body(emb_v_ref, emb_u_ref, pl_ref):
    p = lax.dot_general(
        emb_v_ref[...], emb_u_ref[...], (((1,), (1,)), ((), ())),
        preferred_element_type=jnp.float32, precision=lax.Precision.HIGHEST)
    for ct in range(NCHUNK):
        w = min(128, VOCAB - ct * 128)
        pl_ref[pl.ds(ct * CSTRIDE, VOCAB), pl.ds(0, w)] = (
            p[:, ct * 128:ct * 128 + w])


def _sc_sumexp_body(pl_hbm, ct_hbm, av_hbm, se_hbm, sco_hbm,
                    cen_v, tgt_v, idx_a, idx_b, rows_v, av_v, se_v, sco_v,
                    sem_r, sem_s, sem_a, sem_b):
    cid = lax.axis_index("c")
    sid = lax.axis_index("s")
    wid = sid * NC + cid
    base = wid * RPW
    lanes = lax.broadcasted_iota(jnp.int32, (G,), 0)

    pltpu.sync_copy(ct_hbm.at[pl.ds(base, RPW)], cen_v)
    pltpu.sync_copy(ct_hbm.at[pl.ds(B + base, RPW)], tgt_v)

    ct_off = (lanes & 7) * CSTRIDE
    b_sel = lanes >> 3
    for i in range(8):
        idx_a[pl.ds(i * G, G)] = (
            plsc.load_gather(cen_v, [b_sel + 2 * i]) + ct_off)
        idx_b[pl.ds(i * G, G)] = (
            plsc.load_gather(cen_v, [b_sel + 2 * i + G]) + ct_off)

    def av_copy(g, sem):
        return pltpu.async_copy(
            av_hbm.at[pl.ds((base + g * G) * VOCAB, G * VOCAB)],
            av_v.at[pl.ds(g * G * VOCAB, G * VOCAB)], sem)

    cp_av0 = av_copy(0, sem_a)
    cp_r = pltpu.async_copy(pl_hbm.at[idx_a], rows_v.at[pl.ds(0, 128)], sem_r)
    cp_av1 = av_copy(1, sem_b)
    cp_s = pltpu.async_copy(pl_hbm.at[idx_b], rows_v.at[pl.ds(128, 128)],
                            sem_s)
    cp_av = [cp_av0, cp_av1]

    b8 = [(lanes + g * G) * 8 for g in range(GROUPS)]
    bav = [(lanes + g * G) * VOCAB for g in range(GROUPS)]
    zero = jnp.zeros((G,), jnp.float32)
    accs = []
    for g in range(GROUPS):
        cp_av[g].wait()
        (cp_r if g == 0 else cp_s).wait()

        def step(j0, acc, g=g):
            for u in range(UNROLL):
                j = j0 * UNROLL + u
                iv = plsc.load_gather(av_v, [bav[g] + j])
                vals = plsc.load_gather(
                    rows_v, [b8[g] + (iv >> 7), iv & 127])
                acc = acc + jnp.exp(vals)
            return acc

        accs.append(lax.fori_loop(0, VOCAB // UNROLL, step, zero))

    for g in range(GROUPS):
        se_v[pl.ds(g * G, G)] = accs[g]
        tv = tgt_v[pl.ds(g * G, G)]
        sco_v[pl.ds(g * G, G)] = plsc.load_gather(
            rows_v, [b8[g] + (tv >> 7), tv & 127])
    pltpu.sync_copy(se_v, se_hbm.at[pl.ds(base, RPW)])
    pltpu.sync_copy(sco_v, sco_hbm.at[pl.ds(base, RPW)])


def _tc_final_body(sumexp_ref, scores_ref, o_ref):
    nll = jnp.mean(jnp.log(sumexp_ref[...])) - jnp.mean(scores_ref[...])
    o_ref[...] = nll.reshape(1, 1)


@jax.jit
def kernel(center_words, target_words, all_vocabs, emb_v, emb_u):
    pl_mat = pl.pallas_call(
        _tc_p_body,
        out_shape=jax.ShapeDtypeStruct((PLROWS, 128), jnp.float32),
    )(emb_v, emb_u)

    cen_tgt = jnp.concatenate(
        [center_words.reshape(B), target_words.reshape(B)])

    sumexp, scores = pl.kernel(
        _sc_sumexp_body,
        mesh=plsc.VectorSubcoreMesh(core_axis_name="c", subcore_axis_name="s"),
        out_type=[
            jax.ShapeDtypeStruct((B,), jnp.float32),
            jax.ShapeDtypeStruct((B,), jnp.float32),
        ],
        scratch_types=[
            pltpu.VMEM((RPW,), jnp.int32),
            pltpu.VMEM((RPW,), jnp.int32),
            pltpu.VMEM((128,), jnp.int32),
            pltpu.VMEM((128,), jnp.int32),
            pltpu.VMEM((2 * 128, 128), jnp.float32),
            pltpu.VMEM((RPW * VOCAB,), jnp.int32),
            pltpu.VMEM((RPW,), jnp.float32),
            pltpu.VMEM((RPW,), jnp.float32),
            pltpu.SemaphoreType.DMA,
            pltpu.SemaphoreType.DMA,
            pltpu.SemaphoreType.DMA,
            pltpu.SemaphoreType.DMA,
        ],
        compiler_params=pltpu.CompilerParams(
            use_tc_tiling_on_sc=False, needs_layout_passes=False),
    )(pl_mat, cen_tgt, all_vocabs.reshape(B * VOCAB))

    nll = pl.pallas_call(
        _tc_final_body,
        out_shape=jax.ShapeDtypeStruct((1, 1), jnp.float32),
    )(sumexp.reshape(8, 128), scores.reshape(8, 128))
    return nll[0, 0]


# speedup vs baseline: 1.0123x; 1.0123x over previous
import functools

import jax
import jax.numpy as jnp
from jax import lax
from jax.experimental import pallas as pl
from jax.experimental.pallas import tpu as pltpu
from jax.experimental.pallas import tpu_sc as plsc

VOCAB = 1000
EMB = 64
B = 1024
NCHUNK = 8
CSTRIDE = 1024
PLROWS = NCHUNK * CSTRIDE

NC = 2
NS = 16
NW = NC * NS
RPW = B // NW
G = 16
GROUPS = RPW // G
UNROLL = 8


def _tc_p_body(emb_v_ref, emb_u_ref, pl_ref):
    p = lax.dot_general(
        emb_v_ref[...], emb_u_ref[...], (((1,), (1,)), ((), ())),
        preferred_element_type=jnp.float32, precision=lax.Precision.HIGHEST)
    for ct in range(NCHUNK):
        w = min(128, VOCAB - ct * 128)
        pl_ref[pl.ds(ct * CSTRIDE, VOCAB), pl.ds(0, w)] = (
            p[:, ct * 128:ct * 128 + w])


def _sc_sumexp_body(pl_hbm, ct_hbm, av_hbm, se_hbm, sco_hbm,
                    cen_v, tgt_v, idx_a, idx_b, rows_v, av_v, se_v, sco_v,
                    sem_r, sem_s, sem_a, sem_b):
    cid = lax.axis_index("c")
    sid = lax.axis_index("s")
    wid = sid * NC + cid
    base = wid * RPW
    lanes = lax.broadcasted_iota(jnp.int32, (G,), 0)

    pltpu.sync_copy(ct_hbm.at[pl.ds(base, RPW)], cen_v)
    pltpu.sync_copy(ct_hbm.at[pl.ds(B + base, RPW)], tgt_v)

    ct_off = (lanes & 7) * CSTRIDE
    b_sel = lanes >> 3
    for i in range(8):
        idx_a[pl.ds(i * G, G)] = (
            plsc.load_gather(cen_v, [b_sel + 2 * i]) + ct_off)
        idx_b[pl.ds(i * G, G)] = (
            plsc.load_gather(cen_v, [b_sel + 2 * i + G]) + ct_off)

    def av_copy(g, sem):
        return pltpu.async_copy(
            av_hbm.at[pl.ds((base + g * G) * VOCAB, G * VOCAB)],
            av_v.at[pl.ds(g * G * VOCAB, G * VOCAB)], sem)

    cp_av0 = av_copy(0, sem_a)
    cp_r = pltpu.async_copy(pl_hbm.at[idx_a], rows_v.at[pl.ds(0, 128)], sem_r)
    cp_av1 = av_copy(1, sem_b)
    cp_s = pltpu.async_copy(pl_hbm.at[idx_b], rows_v.at[pl.ds(128, 128)],
                            sem_s)
    cp_av = [cp_av0, cp_av1]

    b8 = [(lanes + g * G) * 8 for g in range(GROUPS)]
    bav = [(lanes + g * G) * VOCAB for g in range(GROUPS)]
    zero = jnp.zeros((G,), jnp.float32)
    NACC = 4
    accs = []
    for g in range(GROUPS):
        cp_av[g].wait()
        (cp_r if g == 0 else cp_s).wait()

        def step(j0, acc, g=g):
            new = list(acc)
            for u in range(UNROLL):
                j = j0 * UNROLL + u
                iv = plsc.load_gather(av_v, [bav[g] + j])
                vals = plsc.load_gather(
                    rows_v, [b8[g] + (iv >> 7), iv & 127])
                new[u % NACC] = new[u % NACC] + jnp.exp(vals)
            return tuple(new)

        parts = lax.fori_loop(0, VOCAB // UNROLL, step, (zero,) * NACC)
        accs.append((parts[0] + parts[1]) + (parts[2] + parts[3]))

    for g in range(GROUPS):
        se_v[pl.ds(g * G, G)] = accs[g]
        tv = tgt_v[pl.ds(g * G, G)]
        sco_v[pl.ds(g * G, G)] = plsc.load_gather(
            rows_v, [b8[g] + (tv >> 7), tv & 127])
    pltpu.sync_copy(se_v, se_hbm.at[pl.ds(base, RPW)])
    pltpu.sync_copy(sco_v, sco_hbm.at[pl.ds(base, RPW)])


def _tc_final_body(sumexp_ref, scores_ref, o_ref):
    nll = jnp.mean(jnp.log(sumexp_ref[...])) - jnp.mean(scores_ref[...])
    o_ref[...] = nll.reshape(1, 1)


@jax.jit
def kernel(center_words, target_words, all_vocabs, emb_v, emb_u):
    pl_mat = pl.pallas_call(
        _tc_p_body,
        out_shape=jax.ShapeDtypeStruct((PLROWS, 128), jnp.float32),
    )(emb_v, emb_u)

    cen_tgt = jnp.concatenate(
        [center_words.reshape(B), target_words.reshape(B)])

    sumexp, scores = pl.kernel(
        _sc_sumexp_body,
        mesh=plsc.VectorSubcoreMesh(core_axis_name="c", subcore_axis_name="s"),
        out_type=[
            jax.ShapeDtypeStruct((B,), jnp.float32),
            jax.ShapeDtypeStruct((B,), jnp.float32),
        ],
        scratch_types=[
            pltpu.VMEM((RPW,), jnp.int32),
            pltpu.VMEM((RPW,), jnp.int32),
            pltpu.VMEM((128,), jnp.int32),
            pltpu.VMEM((128,), jnp.int32),
            pltpu.VMEM((2 * 128, 128), jnp.float32),
            pltpu.VMEM((RPW * VOCAB,), jnp.int32),
            pltpu.VMEM((RPW,), jnp.float32),
            pltpu.VMEM((RPW,), jnp.float32),
            pltpu.SemaphoreType.DMA,
            pltpu.SemaphoreType.DMA,
            pltpu.SemaphoreType.DMA,
            pltpu.SemaphoreType.DMA,
        ],
        compiler_params=pltpu.CompilerParams(
            use_tc_tiling_on_sc=False, needs_layout_passes=False),
    )(pl_mat, cen_tgt, all_vocabs.reshape(B * VOCAB))

    nll = pl.pallas_call(
        _tc_final_body,
        out_shape=jax.ShapeDtypeStruct((1, 1), jnp.float32),
    )(sumexp.reshape(8, 128), scores.reshape(8, 128))
    return nll[0, 0]
